# baseline (speedup 1.0000x reference)
import functools

import jax
import jax.numpy as jnp
from jax import lax
from jax.experimental import pallas as pl
from jax.experimental.pallas import tpu as pltpu
from jax.experimental.pallas import tpu_sc as plsc

_N_PAIRS = 98304
_DIM = 128
_LANES = 16

_info = plsc.get_sparse_core_info()
_NC = _info.num_cores
_NS = _info.num_subcores
_NW = _NC * _NS
_PER_W = _N_PAIRS // _NW
_CHUNK = 128
_NCH = _PER_W // _CHUNK
_PGRP = _LANES * (_LANES + 1)

_LOG1P_COEFFS = (
    3.385588503990178e-08,
    0.9999942730825432,
    -0.4998385694202238,
    0.3315486589665988,
    -0.23982628453301322,
    0.1658229542091144,
    -0.09325222045756972,
    0.0348497958631084,
    -0.006151485802025258,
)


def _sc_partial_loss(src3, tgt3, lab3, nodes_embed, context_nodes_embed):
  mesh = plsc.VectorSubcoreMesh(core_axis_name="c", subcore_axis_name="s")

  @functools.partial(
      pl.kernel,
      mesh=mesh,
      compiler_params=pltpu.CompilerParams(needs_layout_passes=False),
      out_type=jax.ShapeDtypeStruct((_NW, _LANES), jnp.float32),
      scratch_types=[
          pltpu.VMEM((_NCH, _CHUNK), jnp.int32),
          pltpu.VMEM((_NCH, _CHUNK), jnp.int32),
          pltpu.VMEM((_NCH, _CHUNK), jnp.float32),
          pltpu.VMEM((_CHUNK, _DIM), jnp.float32),
          pltpu.VMEM((_CHUNK, _DIM), jnp.float32),
          pltpu.VMEM((_CHUNK, _DIM), jnp.float32),
          pltpu.VMEM((_CHUNK, _DIM), jnp.float32),
          pltpu.VMEM((_LANES,), jnp.float32),
          pltpu.VMEM((_CHUNK // _LANES * _PGRP,), jnp.float32),
          pltpu.SemaphoreType.DMA,
          pltpu.SemaphoreType.DMA,
          pltpu.SemaphoreType.DMA,
          pltpu.SemaphoreType.DMA,
      ],
  )
  def body(src_hbm, tgt_hbm, lab_hbm, nodes_hbm, ctx_hbm, out_hbm,
           sidx, tidx, labv, srows0, srows1, trows0, trows1, accv, pscr,
           sem_s0, sem_s1, sem_t0, sem_t1):
    wid = lax.axis_index("s") * _NC + lax.axis_index("c")
    pltpu.sync_copy(src_hbm.at[wid], sidx)
    pltpu.sync_copy(tgt_hbm.at[wid], tidx)
    pltpu.sync_copy(lab_hbm.at[wid], labv)
    lanes = lax.iota(jnp.int32, _LANES)
    srows = (srows0, srows1)
    trows = (trows0, trows1)
    sem_s = (sem_s0, sem_s1)
    sem_t = (sem_t0, sem_t1)

    def start(j, b):
      pltpu.async_copy(nodes_hbm.at[sidx.at[j]], srows[b], sem_s[b])
      pltpu.async_copy(ctx_hbm.at[tidx.at[j]], trows[b], sem_t[b])

    def wait(b):
      pltpu.make_async_copy(nodes_hbm.at[sidx.at[0]], srows[b], sem_s[b]).wait()
      pltpu.make_async_copy(ctx_hbm.at[tidx.at[0]], trows[b], sem_t[b]).wait()

    def compute(j, b, acc0):
      @plsc.parallel_loop(0, _CHUNK // _LANES, unroll=2, carry=acc0)
      def group_body(g, acc):
        base = g * _PGRP
        for r in range(_LANES):
          row = g * _LANES + r
          p = srows[b][row, pl.ds(0, _LANES)] * trows[b][row, pl.ds(0, _LANES)]
          for k in range(1, _DIM // _LANES):
            p = p + (srows[b][row, pl.ds(k * _LANES, _LANES)]
                     * trows[b][row, pl.ds(k * _LANES, _LANES)])
          pscr[pl.ds(base + r * (_LANES + 1), _LANES)] = p
        stride = base + lanes * (_LANES + 1)
        q = plsc.load_gather(pscr, [stride])
        for d in range(1, _LANES):
          q = q + plsc.load_gather(pscr, [stride + d])
        z = labv[j, pl.ds(g * _LANES, _LANES)] * q
        e = jnp.exp(-jnp.abs(z))
        log1p_e = jnp.float32(_LOG1P_COEFFS[-1])
        for c in reversed(_LOG1P_COEFFS[:-1]):
          log1p_e = log1p_e * e + jnp.float32(c)
        return acc + (jnp.minimum(z, 0.0) - log1p_e)

      return group_body

    start(0, 0)

    def chunk_pair(j2, acc):
      j = j2 * 2
      start(j + 1, 1)
      wait(0)
      acc = compute(j, 0, acc)

      @pl.when(j + 2 < _NCH)
      def _():
        start(j + 2, 0)

      wait(1)
      acc = compute(j + 1, 1, acc)
      return acc

    acc = lax.fori_loop(0, _NCH // 2, chunk_pair,
                        jnp.zeros((_LANES,), jnp.float32))
    accv[...] = acc
    pltpu.sync_copy(accv, out_hbm.at[wid])

  return body(src3, tgt3, lab3, nodes_embed, context_nodes_embed)


def _tc_loss(partials):
  def body(x_ref, o_ref):
    o_ref[0, 0] = -jnp.sum(x_ref[...]) / _N_PAIRS

  out = pl.pallas_call(
      body,
      out_shape=jax.ShapeDtypeStruct((1, 1), jnp.float32),
      out_specs=pl.BlockSpec(memory_space=pltpu.SMEM),
  )(partials)
  return out[0, 0]


def kernel(source_node, target_node, label, nodes_embed, context_nodes_embed):
  src3 = source_node.astype(jnp.int32).reshape(_NW, _NCH, _CHUNK)
  tgt3 = target_node.astype(jnp.int32).reshape(_NW, _NCH, _CHUNK)
  lab3 = label.reshape(_NW, _NCH, _CHUNK)
  partials = _sc_partial_loss(src3, tgt3, lab3, nodes_embed,
                              context_nodes_embed)
  return _tc_loss(partials)
